# baseline (speedup 1.0000x reference)
import functools

import jax
import jax.numpy as jnp
from jax import lax
from jax.experimental import pallas as pl
from jax.experimental.pallas import tpu as pltpu
from jax.experimental.pallas import tpu_sc as plsc

N = 10000
E = 320000
F = 128

NC = 2
NS = 16
L = 16

K = 64
NW = NC * NS
CHT = 160
E_PAD = NW * CHT * K
NBUF = 4
NIDX = 8
ROWS_PER_TILE = 624


def _matmul_body(x_ref, wt_ref, o_ref):
    o_ref[...] = jnp.dot(x_ref[...], wt_ref[...],
                         preferred_element_type=jnp.float32)


def _combine_body(p_ref, b_ref, o_ref):
    o_ref[...] = p_ref[0] + p_ref[1] + b_ref[...]


def _lane_broadcast(wv, l):
    return lax.gather(
        wv, jnp.full((L, 1), l, jnp.int32),
        lax.GatherDimensionNumbers(offset_dims=(),
                                   collapsed_slice_dims=(0,),
                                   start_index_map=(0,)),
        (1,), mode=lax.GatherScatterMode.PROMISE_IN_BOUNDS)


def _sc_edge_body(h_hbm, src_hbm, dst_hbm, w_hbm, part_hbm,
                  acc_sh, src_v, dst_v, w_v, rows_v, sems):
    c = lax.axis_index("c")
    s = lax.axis_index("s")
    wid = c * NS + s
    gsems = [sems.at[i] for i in range(NBUF)]
    ssems = [sems.at[NBUF + i] for i in range(NBUF)]
    isems = [sems.at[2 * NBUF + i] for i in range(NIDX)]

    zeros = jnp.zeros((L,), jnp.float32)

    @pl.loop(0, K)
    def _zero_rows(i):
        for t in range(F // L):
            rows_v[0, i, pl.ds(t * L, L)] = zeros

    row0 = s * ROWS_PER_TILE
    for kk in range(9):
        pltpu.sync_copy(rows_v.at[0],
                        acc_sh.at[pl.ds(row0 + kk * K, K)])
    pltpu.sync_copy(rows_v.at[0].at[pl.ds(0, 48)],
                    acc_sh.at[pl.ds(row0 + 9 * K, 48)])

    @pl.when(s == 0)
    def _zero_tail():
        pltpu.sync_copy(rows_v.at[0].at[pl.ds(0, 16)],
                        acc_sh.at[pl.ds(NS * ROWS_PER_TILE, 16)])

    plsc.subcore_barrier()

    tbase = wid * CHT * K

    def start_idx(j, q):
        base = tbase + j * K
        pltpu.async_copy(src_hbm.at[pl.ds(base, K)], src_v.at[q], isems[q])
        pltpu.async_copy(dst_hbm.at[pl.ds(base, K)], dst_v.at[q], isems[q])
        pltpu.async_copy(w_hbm.at[pl.ds(base, K)], w_v.at[q], isems[q])

    def wait_idx(j, q):
        base = tbase + j * K
        pltpu.make_async_copy(src_hbm.at[pl.ds(base, K)], src_v.at[q],
                              isems[q]).wait()
        pltpu.make_async_copy(dst_hbm.at[pl.ds(base, K)], dst_v.at[q],
                              isems[q]).wait()
        pltpu.make_async_copy(w_hbm.at[pl.ds(base, K)], w_v.at[q],
                              isems[q]).wait()

    def start_gather(q, b):
        pltpu.async_copy(h_hbm.at[src_v.at[q]], rows_v.at[b], gsems[b])

    def wait_gather(q, b):
        pltpu.make_async_copy(h_hbm.at[src_v.at[q]], rows_v.at[b],
                              gsems[b]).wait()

    def start_scatter(q, b):
        pltpu.async_copy(rows_v.at[b], acc_sh.at[dst_v.at[q]],
                         ssems[b], add=True)

    def wait_scatter(q, b):
        pltpu.make_async_copy(rows_v.at[b], acc_sh.at[dst_v.at[q]],
                              ssems[b]).wait()

    def scale(q, b):
        @pl.loop(0, K // L)
        def _scale(g):
            wv = w_v[q, pl.ds(g * L, L)]
            for l in range(L):
                wb = _lane_broadcast(wv, l)
                e = g * L + l
                for t in range(F // L):
                    rows_v[b, e, pl.ds(t * L, L)] = (
                        rows_v[b, e, pl.ds(t * L, L)] * wb)

    for j in range(4):
        start_idx(j, j)
    for j in range(2):
        wait_idx(j, j)
        start_gather(j, j)

    @pl.loop(0, CHT // NIDX)
    def _steps(m):
        for t in range(NIDX):
            j = m * NIDX + t
            b = t % NBUF
            bn = (t + 2) % NBUF
            qn = (t + 2) % NIDX
            wait_gather(t % NIDX, b)
            scale(t % NIDX, b)
            start_scatter(t % NIDX, b)

            @pl.when(j + 2 < CHT)
            def _():
                @pl.when(j >= 2)
                def _():
                    wait_scatter((t - 2) % NIDX, bn)

                @pl.when(j + 4 < CHT)
                def _():
                    start_idx(j + 4, (t + 4) % NIDX)
                wait_idx(j + 2, qn)
                start_gather(qn, bn)

    for b in range(NBUF):
        wait_scatter(0, b)

    plsc.subcore_barrier()
    pltpu.sync_copy(acc_sh.at[pl.ds(row0, ROWS_PER_TILE)],
                    part_hbm.at[c, pl.ds(row0, ROWS_PER_TILE)])

    @pl.when(s == 0)
    def _flush_tail():
        pltpu.sync_copy(acc_sh.at[pl.ds(NS * ROWS_PER_TILE, 16)],
                        part_hbm.at[c, pl.ds(NS * ROWS_PER_TILE, 16)])


def kernel(x, edge_index, edge_weight, node_lock, W, b):
    del node_lock
    h = pl.pallas_call(
        _matmul_body,
        grid=(10,),
        in_specs=[pl.BlockSpec((N // 10, F), lambda i: (i, 0)),
                  pl.BlockSpec((F, F), lambda i: (0, 0))],
        out_specs=pl.BlockSpec((N // 10, F), lambda i: (i, 0)),
        out_shape=jax.ShapeDtypeStruct((N, F), jnp.float32),
    )(x, W.T)

    pad = E_PAD - E
    src1 = jnp.pad(edge_index[0], (0, pad))
    dst1 = jnp.pad(edge_index[1], (0, pad))
    w1 = jnp.pad(edge_weight, (0, pad))

    mesh = plsc.VectorSubcoreMesh(core_axis_name="c", subcore_axis_name="s",
                                  num_cores=NC, num_subcores=NS)
    sc_edges = pl.kernel(
        _sc_edge_body,
        out_type=jax.ShapeDtypeStruct((NC, N, F), jnp.float32),
        mesh=mesh,
        scratch_types=[
            pltpu.VMEM_SHARED((N, F), jnp.float32),
            pltpu.VMEM((NIDX, K), jnp.int32),
            pltpu.VMEM((NIDX, K), jnp.int32),
            pltpu.VMEM((NIDX, K), jnp.float32),
            pltpu.VMEM((NBUF, K, F), jnp.float32),
            pltpu.SemaphoreType.DMA((2 * NBUF + NIDX,)),
        ],
    )
    part = sc_edges(h, src1, dst1, w1)

    out = pl.pallas_call(
        _combine_body,
        grid=(10,),
        in_specs=[pl.BlockSpec((NC, N // 10, F), lambda i: (0, i, 0)),
                  pl.BlockSpec((1, F), lambda i: (0, 0))],
        out_specs=pl.BlockSpec((N // 10, F), lambda i: (i, 0)),
        out_shape=jax.ShapeDtypeStruct((N, F), jnp.float32),
    )(part, b.reshape(1, F))

    return (out, edge_index, edge_weight)
